# baseline (speedup 1.0000x reference)
import functools

import jax
import jax.numpy as jnp
from jax import lax
from jax.experimental import pallas as pl
from jax.experimental.pallas import tpu as pltpu
from jax.experimental.pallas import tpu_sc as plsc

_CHUNK = 128


_BT = 4096


def _prep_body(a_ref, ao_ref):
    a = a_ref[...]
    ao_ref[...] = jnp.concatenate([a[:, :_BT].T, a[:, _BT:].T], axis=1)


def _pack_table(table):
    v, h = table.shape
    grid = (v + 2 * _BT - 1) // (2 * _BT)
    vp = grid * _BT
    return pl.pallas_call(
        _prep_body,
        grid=(grid,),
        in_specs=[pl.BlockSpec((h, 2 * _BT), lambda i: (0, i))],
        out_specs=pl.BlockSpec((_BT, 2 * h), lambda i: (i, 0)),
        out_shape=jax.ShapeDtypeStruct((vp, 2 * h), jnp.float32),
    )(table.T)


def _sc_gather_body(nc, n_chunks, b_per_w, which,
                    tab, idx, out, idx_v, rows_v, sem):
    wid = lax.axis_index("s") * nc + lax.axis_index("c")
    pltpu.sync_copy(idx.at[wid], idx_v)
    copies = []
    for j in range(n_chunks):
        copies.append(pltpu.async_copy(
            tab.at[idx_v.at[which * n_chunks + j]],
            rows_v.at[pl.ds(j * _CHUNK, _CHUNK)], sem))
    for c in copies:
        c.wait()
    pltpu.sync_copy(rows_v, out.at[pl.ds(wid * b_per_w, b_per_w)])


def _sc_gather_one(tab_pairs, idx_all, n, which):
    w = tab_pairs.shape[1]
    info = plsc.get_sparse_core_info()
    nc, ns = info.num_cores, info.num_subcores
    nw = nc * ns
    b_per_w = n // nw
    assert b_per_w * nw == n and b_per_w % _CHUNK == 0
    n_chunks = b_per_w // _CHUNK
    mesh = plsc.VectorSubcoreMesh(core_axis_name="c", subcore_axis_name="s")
    f = pl.kernel(
        functools.partial(_sc_gather_body, nc, n_chunks, b_per_w, which),
        out_type=jax.ShapeDtypeStruct((n, w), jnp.float32),
        mesh=mesh,
        scratch_types=[
            pltpu.VMEM((2 * n_chunks, _CHUNK), jnp.int32),
            pltpu.VMEM((b_per_w, w), jnp.float32),
            pltpu.SemaphoreType.DMA,
        ],
    )
    return f(tab_pairs, idx_all)


def _tc_dense_body(act_ref, res_ref, num_ref, aid_ref, rid_ref,
                   nlg_ref, nlb_ref, w1_ref, b1_ref, mlg_ref, mlb_ref,
                   wg_ref, bg_ref, wb_ref, bb_ref, wpc_ref, wpn_ref, bp_ref,
                   plg_ref, plb_ref, out_ref):
    eps = 1e-5
    num = num_ref[...]
    nf = jnp.log(1.0 + jnp.maximum(num, 0.0))
    mu = jnp.mean(nf, axis=-1, keepdims=True)
    var = jnp.mean((nf - mu) ** 2, axis=-1, keepdims=True)
    nf = (nf - mu) * lax.rsqrt(var + eps) * nlg_ref[...] + nlb_ref[...]
    hid = jnp.dot(nf, w1_ref[...], preferred_element_type=jnp.float32)
    hid = jnp.maximum(hid + b1_ref[...], 0.0)
    mu = jnp.mean(hid, axis=-1, keepdims=True)
    var = jnp.mean((hid - mu) ** 2, axis=-1, keepdims=True)
    num_emb = (hid - mu) * lax.rsqrt(var + eps) * mlg_ref[...] + mlb_ref[...]
    g_in = jnp.dot(num_emb, wg_ref[...], preferred_element_type=jnp.float32)
    gamma = 1.0 / (1.0 + jnp.exp(-(g_in + bg_ref[...])))
    beta = jnp.dot(num_emb, wb_ref[...],
                   preferred_element_type=jnp.float32) + bb_ref[...]
    aid = aid_ref[...]
    rid = rid_ref[...]
    a_pairs = act_ref[...]
    r_pairs = res_ref[...]
    h = a_pairs.shape[1] // 2
    a_hi = ((aid >> 12) & 1) == 1
    r_hi = ((rid >> 12) & 1) == 1
    act_emb = jnp.where(a_hi, a_pairs[:, h:], a_pairs[:, :h])
    res_emb = jnp.where(r_hi, r_pairs[:, h:], r_pairs[:, :h])
    cat = jnp.concatenate([act_emb, res_emb], axis=-1)
    cat_mod = cat * gamma + beta
    is_pad = (aid == 0) & (rid == 0)
    cat_mod = jnp.where(is_pad, 0.0, cat_mod)
    num_emb = jnp.where(is_pad, 0.0, num_emb)
    comb = (jnp.dot(cat_mod, wpc_ref[...], preferred_element_type=jnp.float32)
            + jnp.dot(num_emb, wpn_ref[...], preferred_element_type=jnp.float32)
            + bp_ref[...])
    comb = jnp.maximum(comb, 0.0)
    mu = jnp.mean(comb, axis=-1, keepdims=True)
    var = jnp.mean((comb - mu) ** 2, axis=-1, keepdims=True)
    out_ref[...] = (comb - mu) * lax.rsqrt(var + eps) * plg_ref[...] + plb_ref[...]


def kernel(activities, resources, num_arr, act_table, res_table,
           num_ln_g, num_ln_b, W1, b1, mlp_ln_g, mlp_ln_b,
           Wg, bg, Wb, bb, Wp, bp, proj_ln_g, proj_ln_b):
    n = activities.shape[0]
    v, h = act_table.shape
    d = W1.shape[1]
    f = num_arr.shape[1]
    acts = activities.astype(jnp.int32)
    ress = resources.astype(jnp.int32)

    nw = 32
    n_chunks = n // nw // _CHUNK
    pair_row = lambda i: ((i >> 13) << 12) | (i & (_BT - 1))
    a3 = pair_row(acts).reshape(nw, n_chunks, _CHUNK)
    r3 = pair_row(ress).reshape(nw, n_chunks, _CHUNK)
    idx_all = jnp.concatenate([a3, r3], axis=1)

    act_pairs = _pack_table(act_table)
    a_emb2 = _sc_gather_one(act_pairs, idx_all, n, 0)
    res_pairs = _pack_table(res_table)
    r_emb2 = _sc_gather_one(res_pairs, idx_all, n, 1)

    bn = 2048
    nblk = n // bn
    row_spec = lambda w: pl.BlockSpec((bn, w), lambda i: (i, 0))
    full_spec = lambda s: pl.BlockSpec(s, lambda i: tuple(0 for _ in s))
    out = pl.pallas_call(
        _tc_dense_body,
        grid=(nblk,),
        in_specs=[
            row_spec(2 * h), row_spec(2 * h), row_spec(f),
            row_spec(1), row_spec(1),
            full_spec((1, f)), full_spec((1, f)),
            full_spec((f, d)), full_spec((1, d)),
            full_spec((1, d)), full_spec((1, d)),
            full_spec((d, d)), full_spec((1, d)),
            full_spec((d, d)), full_spec((1, d)),
            full_spec((d, d)), full_spec((d, d)), full_spec((1, d)),
            full_spec((1, d)), full_spec((1, d)),
        ],
        out_specs=row_spec(d),
        out_shape=jax.ShapeDtypeStruct((n, d), jnp.float32),
    )(
        a_emb2, r_emb2, num_arr,
        acts.reshape(n, 1), ress.reshape(n, 1),
        num_ln_g.reshape(1, f), num_ln_b.reshape(1, f),
        W1, b1.reshape(1, d),
        mlp_ln_g.reshape(1, d), mlp_ln_b.reshape(1, d),
        Wg, bg.reshape(1, d),
        Wb, bb.reshape(1, d),
        Wp[:d], Wp[d:], bp.reshape(1, d),
        proj_ln_g.reshape(1, d), proj_ln_b.reshape(1, d),
    )
    return out


# speedup vs baseline: 1.0710x; 1.0710x over previous
import functools

import jax
import jax.numpy as jnp
from jax import lax
from jax.experimental import pallas as pl
from jax.experimental.pallas import tpu as pltpu
from jax.experimental.pallas import tpu_sc as plsc

_CHUNK = 128


_BT = 4096


def _prep_body(a_ref, r_ref, ao_ref, ro_ref):
    a = a_ref[...]
    r = r_ref[...]
    ao_ref[...] = jnp.concatenate([a[:, :_BT].T, a[:, _BT:].T], axis=1)
    ro_ref[...] = jnp.concatenate([r[:, :_BT].T, r[:, _BT:].T], axis=1)


def _pack_tables(act_table, res_table):
    v, h = act_table.shape
    grid = (v + 2 * _BT - 1) // (2 * _BT)
    vp = grid * _BT
    return pl.pallas_call(
        _prep_body,
        grid=(grid,),
        in_specs=[pl.BlockSpec((h, 2 * _BT), lambda i: (0, i)),
                  pl.BlockSpec((h, 2 * _BT), lambda i: (0, i))],
        out_specs=[pl.BlockSpec((_BT, 2 * h), lambda i: (i, 0)),
                   pl.BlockSpec((_BT, 2 * h), lambda i: (i, 0))],
        out_shape=[jax.ShapeDtypeStruct((vp, 2 * h), jnp.float32),
                   jax.ShapeDtypeStruct((vp, 2 * h), jnp.float32)],
    )(act_table.T, res_table.T)


def _sc_gather_body(nc, n_chunks, b_per_w,
                    act_tab, res_tab, idx, act_out, res_out,
                    idx_v, rows_a, rows_r, sem_a, sem_r):
    wid = lax.axis_index("s") * nc + lax.axis_index("c")
    pltpu.sync_copy(idx.at[wid], idx_v)
    hb = n_chunks // 2
    for p in range(2):
        copies = []
        for j in range(hb):
            copies.append(pltpu.async_copy(
                act_tab.at[idx_v.at[p * hb + j]],
                rows_a.at[pl.ds(j * _CHUNK, _CHUNK)], sem_a))
            copies.append(pltpu.async_copy(
                res_tab.at[idx_v.at[n_chunks + p * hb + j]],
                rows_r.at[pl.ds(j * _CHUNK, _CHUNK)], sem_r))
        for c in copies:
            c.wait()
        base = wid * b_per_w + p * hb * _CHUNK
        pltpu.sync_copy(rows_a, act_out.at[pl.ds(base, hb * _CHUNK)])
        pltpu.sync_copy(rows_r, res_out.at[pl.ds(base, hb * _CHUNK)])


def _sc_gather(act_pairs, res_pairs, idx_all, n, dtype):
    w = act_pairs.shape[1]
    info = plsc.get_sparse_core_info()
    nc, ns = info.num_cores, info.num_subcores
    nw = nc * ns
    b_per_w = n // nw
    assert b_per_w * nw == n and b_per_w % _CHUNK == 0
    n_chunks = b_per_w // _CHUNK
    mesh = plsc.VectorSubcoreMesh(core_axis_name="c", subcore_axis_name="s")
    f = pl.kernel(
        functools.partial(_sc_gather_body, nc, n_chunks, b_per_w),
        out_type=(jax.ShapeDtypeStruct((n, w), dtype),
                  jax.ShapeDtypeStruct((n, w), dtype)),
        mesh=mesh,
        scratch_types=[
            pltpu.VMEM((2 * n_chunks, _CHUNK), jnp.int32),
            pltpu.VMEM((b_per_w // 2, w), dtype),
            pltpu.VMEM((b_per_w // 2, w), dtype),
            pltpu.SemaphoreType.DMA,
            pltpu.SemaphoreType.DMA,
        ],
    )
    return f(act_pairs, res_pairs, idx_all)


def _tc_dense_body(act_ref, res_ref, num_ref, aid_ref, rid_ref,
                   nlg_ref, nlb_ref, w1_ref, b1_ref, mlg_ref, mlb_ref,
                   wg_ref, bg_ref, wb_ref, bb_ref, wpc_ref, wpn_ref, bp_ref,
                   plg_ref, plb_ref, out_ref):
    eps = 1e-5
    num = num_ref[...]
    nf = jnp.log(1.0 + jnp.maximum(num, 0.0))
    mu = jnp.mean(nf, axis=-1, keepdims=True)
    var = jnp.mean((nf - mu) ** 2, axis=-1, keepdims=True)
    nf = (nf - mu) * lax.rsqrt(var + eps) * nlg_ref[...] + nlb_ref[...]
    hid = jnp.dot(nf, w1_ref[...], preferred_element_type=jnp.float32)
    hid = jnp.maximum(hid + b1_ref[...], 0.0)
    mu = jnp.mean(hid, axis=-1, keepdims=True)
    var = jnp.mean((hid - mu) ** 2, axis=-1, keepdims=True)
    num_emb = (hid - mu) * lax.rsqrt(var + eps) * mlg_ref[...] + mlb_ref[...]
    g_in = jnp.dot(num_emb, wg_ref[...], preferred_element_type=jnp.float32)
    gamma = 1.0 / (1.0 + jnp.exp(-(g_in + bg_ref[...])))
    beta = jnp.dot(num_emb, wb_ref[...],
                   preferred_element_type=jnp.float32) + bb_ref[...]
    aid = aid_ref[...]
    rid = rid_ref[...]
    a_pairs = act_ref[...]
    r_pairs = res_ref[...]
    h = a_pairs.shape[1] // 2
    a_hi = ((aid >> 12) & 1) == 1
    r_hi = ((rid >> 12) & 1) == 1
    act_emb = jnp.where(a_hi, a_pairs[:, h:], a_pairs[:, :h])
    res_emb = jnp.where(r_hi, r_pairs[:, h:], r_pairs[:, :h])
    cat = jnp.concatenate([act_emb, res_emb], axis=-1)
    cat_mod = cat * gamma + beta
    is_pad = (aid == 0) & (rid == 0)
    cat_mod = jnp.where(is_pad, 0.0, cat_mod)
    num_emb = jnp.where(is_pad, 0.0, num_emb)
    comb = (jnp.dot(cat_mod, wpc_ref[...], preferred_element_type=jnp.float32)
            + jnp.dot(num_emb, wpn_ref[...], preferred_element_type=jnp.float32)
            + bp_ref[...])
    comb = jnp.maximum(comb, 0.0)
    mu = jnp.mean(comb, axis=-1, keepdims=True)
    var = jnp.mean((comb - mu) ** 2, axis=-1, keepdims=True)
    out_ref[...] = (comb - mu) * lax.rsqrt(var + eps) * plg_ref[...] + plb_ref[...]


def kernel(activities, resources, num_arr, act_table, res_table,
           num_ln_g, num_ln_b, W1, b1, mlp_ln_g, mlp_ln_b,
           Wg, bg, Wb, bb, Wp, bp, proj_ln_g, proj_ln_b):
    n = activities.shape[0]
    v, h = act_table.shape
    d = W1.shape[1]
    f = num_arr.shape[1]
    acts = activities.astype(jnp.int32)
    ress = resources.astype(jnp.int32)

    nw = 32
    n_chunks = n // nw // _CHUNK
    pair_row = lambda i: ((i >> 13) << 12) | (i & (_BT - 1))
    a3 = pair_row(acts).reshape(nw, n_chunks, _CHUNK)
    r3 = pair_row(ress).reshape(nw, n_chunks, _CHUNK)
    idx_all = jnp.concatenate([a3, r3], axis=1)

    act_pairs, res_pairs = _pack_tables(act_table, res_table)
    a_emb2, r_emb2 = _sc_gather(act_pairs, res_pairs, idx_all, n, jnp.float32)

    bn = 2048
    nblk = n // bn
    row_spec = lambda w: pl.BlockSpec((bn, w), lambda i: (i, 0))
    full_spec = lambda s: pl.BlockSpec(s, lambda i: tuple(0 for _ in s))
    out = pl.pallas_call(
        _tc_dense_body,
        grid=(nblk,),
        in_specs=[
            row_spec(2 * h), row_spec(2 * h), row_spec(f),
            row_spec(1), row_spec(1),
            full_spec((1, f)), full_spec((1, f)),
            full_spec((f, d)), full_spec((1, d)),
            full_spec((1, d)), full_spec((1, d)),
            full_spec((d, d)), full_spec((1, d)),
            full_spec((d, d)), full_spec((1, d)),
            full_spec((d, d)), full_spec((d, d)), full_spec((1, d)),
            full_spec((1, d)), full_spec((1, d)),
        ],
        out_specs=row_spec(d),
        out_shape=jax.ShapeDtypeStruct((n, d), jnp.float32),
    )(
        a_emb2, r_emb2, num_arr,
        acts.reshape(n, 1), ress.reshape(n, 1),
        num_ln_g.reshape(1, f), num_ln_b.reshape(1, f),
        W1, b1.reshape(1, d),
        mlp_ln_g.reshape(1, d), mlp_ln_b.reshape(1, d),
        Wg, bg.reshape(1, d),
        Wb, bb.reshape(1, d),
        Wp[:d], Wp[d:], bp.reshape(1, d),
        proj_ln_g.reshape(1, d), proj_ln_b.reshape(1, d),
    )
    return out
